# Initial kernel scaffold; baseline (speedup 1.0000x reference)
import jax
import jax.numpy as jnp
from jax import lax
from jax.experimental import pallas as pl
from jax.experimental.pallas import tpu as pltpu

_MB, _NB = 100, 100
_N = _MB * _NB
_E = 128
_U = 256
_H = 100
_RN = 2000
_NRB = _N // _RN
_NP = _N + 2 * _H


def _gnn_body(xsa_ref, in_W_ref, in_b_ref, Wcol_ref, col_b1_ref, Wrow_ref,
              row_b1_ref, W2c_ref, W2r_ref, msb_ref, eW1_ref, eb1_ref,
              eW2_ref, eb2_ref, ro_row_ref, ro2_ref, rdc_ref,
              out_ref, buf_ref):
    s = pl.program_id(1)
    rb = pl.program_id(2)
    base = rb * _RN
    p_in = (s + 1) % 2
    p_out = s % 2

    def write_h(h):
        buf_ref[p_out, pl.ds(base + _H, _RN), :] = h

        @pl.when(rb == 0)
        def _():
            buf_ref[p_out, pl.ds(0, _H), :] = h[0:_H]

        @pl.when(rb == _NRB - 1)
        def _():
            buf_ref[p_out, pl.ds(_N + _H, _H), :] = h[_RN - _H:_RN]

    @pl.when(s == 0)
    def _embed():
        x = xsa_ref[0]
        h0 = (x[:, 0:1] * in_W_ref[0:1, :] + x[:, 1:2] * in_W_ref[1:2, :]
              + in_b_ref[...])
        write_h(h0)

    @pl.when(s > 0)
    def _layer():
        S = buf_ref[p_in, pl.ds(base, _RN + 2 * _H), :]
        cb1 = col_b1_ref[...]
        rb1 = row_b1_ref[...]

        Ucol = jnp.dot(S, Wcol_ref[...], preferred_element_type=jnp.float32)
        Pc = Ucol[_H:_H + _RN, 0:_U]
        Yc = Ucol[:, _U:2 * _U]
        A_ud = (jnp.maximum(Pc + Yc[0:_RN] + cb1, 0.0)
                + jnp.maximum(Pc + Yc[2 * _H:2 * _H + _RN] + cb1, 0.0))

        Urow = jnp.dot(S, Wrow_ref[...], preferred_element_type=jnp.float32)
        Pr = Urow[_H:_H + _RN, 0:_U]
        Yr = Urow[:, _U:2 * _U]
        y_self = Yr[_H:_H + _RN]
        i0 = lax.broadcasted_iota(jnp.int32, (_RN, _U), 0) % _NB
        y_left = jnp.where(i0 == 0, y_self, Yr[_H - 1:_H - 1 + _RN])
        y_right = jnp.where(i0 == _NB - 1, y_self, Yr[_H + 1:_H + 1 + _RN])
        A_lr = (jnp.maximum(Pr + y_left + rb1, 0.0)
                + jnp.maximum(Pr + y_right + rb1, 0.0))

        m = (jnp.dot(A_ud, W2c_ref[...], preferred_element_type=jnp.float32)
             + jnp.dot(A_lr, W2r_ref[...], preferred_element_type=jnp.float32)
             + msb_ref[...])

        hs = S[_H:_H + _RN, :]
        X = jnp.concatenate([hs, m], axis=1)
        g = jnp.maximum(
            jnp.dot(X, eW1_ref[...], preferred_element_type=jnp.float32)
            + eb1_ref[...], 0.0)
        hn = (jnp.dot(g, eW2_ref[...], preferred_element_type=jnp.float32)
              + eb2_ref[...])
        write_h(hn)

        @pl.when(s == 3)
        def _readout():
            part = jnp.sum(hn * ro2_ref[...] * ro_row_ref[...])

            @pl.when(rb == 0)
            def _():
                out_ref[0, 0, 0] = rdc_ref[0, 0] + part

            @pl.when(rb > 0)
            def _():
                out_ref[0, 0, 0] = out_ref[0, 0, 0] + part


def kernel(state, actor, up, down, left, right, in_W, in_b,
           row_W1, row_b1, row_W2, row_b2,
           col_W1, col_b1, col_W2, col_b2,
           emb_W1, emb_b1, emb_W2, emb_b2,
           ro_W, ro_b, ro2_W, ro2_b):
    B = state.shape[0]
    xsa = jnp.stack([state.reshape(B, _N), actor.reshape(B, _N)], axis=-1)

    Wcol = jnp.concatenate([col_W1[:_E], col_W1[_E:]], axis=1)
    Wrow = jnp.concatenate([row_W1[:_E], row_W1[_E:]], axis=1)
    msb = (2.0 * (col_b2 + row_b2)).reshape(1, _E)
    rdc = (ro_b[0] * jnp.sum(ro2_W) + ro2_b[0]).reshape(1, 1)
    ro_row = ro_W.reshape(1, _E)

    grid = (B, 4, _NRB)

    def full(shape):
        return pl.BlockSpec(shape, lambda b, s, r: (0,) * len(shape))

    out = pl.pallas_call(
        _gnn_body,
        grid=grid,
        in_specs=[
            pl.BlockSpec((1, _RN, 2), lambda b, s, r: (b, r, 0)),
            full((2, _E)),
            full((1, _E)),
            full((_E, 2 * _U)),
            full((1, _U)),
            full((_E, 2 * _U)),
            full((1, _U)),
            full((_U, _E)),
            full((_U, _E)),
            full((1, _E)),
            full((2 * _E, _U)),
            full((1, _U)),
            full((_U, _E)),
            full((1, _E)),
            full((1, _E)),
            pl.BlockSpec((_RN, 1), lambda b, s, r: (r, 0)),
            full((1, 1)),
        ],
        out_specs=pl.BlockSpec((1, 1, 1), lambda b, s, r: (b, 0, 0)),
        out_shape=jax.ShapeDtypeStruct((B, 1, 1), jnp.float32),
        scratch_shapes=[pltpu.VMEM((2, _NP, _E), jnp.float32)],
        compiler_params=pltpu.CompilerParams(
            dimension_semantics=("arbitrary", "arbitrary", "arbitrary")),
    )(xsa, in_W, in_b.reshape(1, _E), Wcol, col_b1.reshape(1, _U),
      Wrow, row_b1.reshape(1, _U), col_W2, row_W2, msb,
      emb_W1, emb_b1.reshape(1, _U), emb_W2, emb_b2.reshape(1, _E),
      ro_row, ro2_W, rdc)
    return out.reshape(B, 1)


# speedup vs baseline: 1.8701x; 1.8701x over previous
import jax
import jax.numpy as jnp
from jax import lax
from jax.experimental import pallas as pl
from jax.experimental.pallas import tpu as pltpu

_MB, _NB = 100, 100
_N = _MB * _NB
_E = 128
_U = 256
_H = 100
_RN = 2000
_NRB = _N // _RN
_NP = _N + 2 * _H


def _gnn_body(xsa_ref, in_W_ref, in_b_ref, Wcol_ref, col_b1_ref, Wrow_ref,
              row_b1_ref, W2c_ref, W2r_ref, msb_ref, eW1_ref, eb1_ref,
              eW2_ref, eb2_ref, ro_row_ref, ro2_ref, rdc_ref,
              out_ref, buf_ref):
    s = pl.program_id(1)
    rb = pl.program_id(2)
    base = rb * _RN
    p_in = (s + 1) % 2
    p_out = s % 2

    def write_h(h):
        buf_ref[p_out, pl.ds(base + _H, _RN), :] = h

        @pl.when(rb == 0)
        def _():
            buf_ref[p_out, pl.ds(0, _H), :] = h[0:_H]

        @pl.when(rb == _NRB - 1)
        def _():
            buf_ref[p_out, pl.ds(_N + _H, _H), :] = h[_RN - _H:_RN]

    @pl.when(s == 0)
    def _embed():
        x = xsa_ref[0]
        h0 = (x[:, 0:1] * in_W_ref[0:1, :] + x[:, 1:2] * in_W_ref[1:2, :]
              + in_b_ref[...])
        write_h(h0)

    @pl.when(s > 0)
    def _layer():
        S = buf_ref[p_in, pl.ds(base, _RN + 2 * _H), :]
        cb1 = col_b1_ref[...]
        rb1 = row_b1_ref[...]

        Ucol = jnp.dot(S, Wcol_ref[...], preferred_element_type=jnp.float32)
        Pc = Ucol[_H:_H + _RN, 0:_U]
        Yc = Ucol[:, _U:2 * _U]
        A_ud = (jnp.maximum(Pc + Yc[0:_RN] + cb1, 0.0)
                + jnp.maximum(Pc + Yc[2 * _H:2 * _H + _RN] + cb1, 0.0))

        Urow = jnp.dot(S, Wrow_ref[...], preferred_element_type=jnp.float32)
        Pr = Urow[_H:_H + _RN, 0:_U]
        Yr = Urow[:, _U:2 * _U]
        y_self = Yr[_H:_H + _RN]
        i0 = lax.broadcasted_iota(jnp.int32, (_RN, _U), 0) % _NB
        y_left = jnp.where(i0 == 0, y_self, Yr[_H - 1:_H - 1 + _RN])
        y_right = jnp.where(i0 == _NB - 1, y_self, Yr[_H + 1:_H + 1 + _RN])
        A_lr = (jnp.maximum(Pr + y_left + rb1, 0.0)
                + jnp.maximum(Pr + y_right + rb1, 0.0))

        m = (jnp.dot(A_ud, W2c_ref[...], preferred_element_type=jnp.float32)
             + jnp.dot(A_lr, W2r_ref[...], preferred_element_type=jnp.float32)
             + msb_ref[...])

        hs = S[_H:_H + _RN, :]
        X = jnp.concatenate([hs, m], axis=1)
        g = jnp.maximum(
            jnp.dot(X, eW1_ref[...], preferred_element_type=jnp.float32)
            + eb1_ref[...], 0.0)
        hn = (jnp.dot(g, eW2_ref[...], preferred_element_type=jnp.float32)
              + eb2_ref[...])
        write_h(hn)

        @pl.when(s == 3)
        def _readout():
            part = jnp.sum(hn * ro2_ref[...] * ro_row_ref[...])

            @pl.when(rb == 0)
            def _():
                out_ref[0] = rdc_ref[...] + part

            @pl.when(rb > 0)
            def _():
                out_ref[0] = out_ref[0] + part


def kernel(state, actor, up, down, left, right, in_W, in_b,
           row_W1, row_b1, row_W2, row_b2,
           col_W1, col_b1, col_W2, col_b2,
           emb_W1, emb_b1, emb_W2, emb_b2,
           ro_W, ro_b, ro2_W, ro2_b):
    B = state.shape[0]
    xsa = jnp.stack([state.reshape(B, _N), actor.reshape(B, _N)], axis=-1)

    Wcol = jnp.concatenate([col_W1[:_E], col_W1[_E:]], axis=1)
    Wrow = jnp.concatenate([row_W1[:_E], row_W1[_E:]], axis=1)
    msb = (2.0 * (col_b2 + row_b2)).reshape(1, _E)
    rdc = (ro_b[0] * jnp.sum(ro2_W) + ro2_b[0]).reshape(1, 1)
    ro_row = ro_W.reshape(1, _E)

    grid = (B, 4, _NRB)

    def full(shape):
        return pl.BlockSpec(shape, lambda b, s, r: (0,) * len(shape))

    out = pl.pallas_call(
        _gnn_body,
        grid=grid,
        in_specs=[
            pl.BlockSpec((1, _RN, 2), lambda b, s, r: (b, r, 0)),
            full((2, _E)),
            full((1, _E)),
            full((_E, 2 * _U)),
            full((1, _U)),
            full((_E, 2 * _U)),
            full((1, _U)),
            full((_U, _E)),
            full((_U, _E)),
            full((1, _E)),
            full((2 * _E, _U)),
            full((1, _U)),
            full((_U, _E)),
            full((1, _E)),
            full((1, _E)),
            pl.BlockSpec((_RN, 1), lambda b, s, r: (r, 0)),
            full((1, 1)),
        ],
        out_specs=pl.BlockSpec((1, 1, 1), lambda b, s, r: (b, 0, 0)),
        out_shape=jax.ShapeDtypeStruct((B, 1, 1), jnp.float32),
        scratch_shapes=[pltpu.VMEM((2, _NP, _E), jnp.float32)],
        compiler_params=pltpu.CompilerParams(
            dimension_semantics=("arbitrary", "arbitrary", "arbitrary")),
    )(xsa, in_W, in_b.reshape(1, _E), Wcol, col_b1.reshape(1, _U),
      Wrow, row_b1.reshape(1, _U), col_W2, row_W2, msb,
      emb_W1, emb_b1.reshape(1, _U), emb_W2, emb_b2.reshape(1, _E),
      ro_row, ro2_W, rdc)
    return out.reshape(B, 1)
